# baseline (speedup 1.0000x reference)
import functools
import math

import jax
import jax.numpy as jnp
from jax import lax
from jax.experimental import pallas as pl
from jax.experimental.pallas import tpu as pltpu
from jax.experimental.pallas import tpu_sc as plsc

_D = 64
_SCALE = math.sqrt(_D)
_V = 1000000
_B = 16384
_S = 50
_L = 16
_BLK = 128
_NBB = _B // _BLK
_TC_CB = 4096


def _tc_transpose_body(t_ref, out_ref):
    eye = jnp.eye(_D, dtype=jnp.float32) * _SCALE
    out_ref[:, :_D] = lax.dot_general(
        t_ref[...], eye, (((0,), (0,)), ((), ())),
        preferred_element_type=jnp.float32,
    )


def _build_t2(table_t):
    grid = (_V + _TC_CB - 1) // _TC_CB
    return pl.pallas_call(
        _tc_transpose_body,
        grid=(grid,),
        in_specs=[pl.BlockSpec((_D, _TC_CB), lambda j: (0, j))],
        out_specs=pl.BlockSpec((_TC_CB, 128), lambda j: (j, 0)),
        out_shape=jax.ShapeDtypeStruct((_V, 128), jnp.float32),
    )(table_t)


_NG = 4
_NO = 3
_NU = 200


def _sc_body(x_t_hbm, t2_hbm, out_hbm, idx_v, g_bufs, stags, gsems, osems):
    ncores = 2
    wid = lax.axis_index("s") * ncores + lax.axis_index("c")
    bb0 = wid * 4

    for t in range(28):
        bi, stile = t // 7, t % 7
        rows = 8 if stile < 6 else 2
        pltpu.async_copy(
            x_t_hbm.at[pl.ds(stile * 8, rows),
                       pl.ds((bb0 + bi) * _BLK, _BLK)],
            idx_v.at[t, pl.ds(0, rows), :],
            gsems[0],
        )
    for t in range(28):
        rows = 8 if (t % 7) != 6 else 2
        pltpu.make_async_copy(
            x_t_hbm.at[pl.ds(0, rows), pl.ds(0, _BLK)],
            idx_v.at[0, pl.ds(0, rows), :],
            gsems[0],
        ).wait()

    iota = lax.iota(jnp.int32, _L)
    rowvecs = [iota + (g * _L) for g in range(8)]

    def unit_coords(n):
        bi = n // _S
        s = n % _S
        t = bi * 7 + s // 8
        si = s % 8
        return bi, s, t, si

    def fire(n, b):
        _, _, t, si = unit_coords(n)
        pltpu.async_copy(t2_hbm.at[idx_v.at[t, si]], g_bufs[b], gsems[b])

    def process(n, b, ob):
        bi, s, _, _ = unit_coords(n)
        pltpu.make_async_copy(
            t2_hbm.at[pl.ds(0, _BLK)], g_bufs[b], gsems[b]
        ).wait()

        @plsc.parallel_loop(0, _D, unroll=8)
        def _col(k):
            colvec = jnp.full((_L,), 0, jnp.int32) + k
            for g in range(8):
                stags[ob][k, pl.ds(g * _L, _L)] = plsc.load_gather(
                    g_bufs[b], [rowvecs[g], colvec]
                )

        pltpu.async_copy(
            stags[ob], out_hbm.at[s, :, pl.ds((bb0 + bi) * _BLK, _BLK)],
            osems[ob],
        )

    def wait_out(ob):
        pltpu.make_async_copy(
            t2_hbm.at[pl.ds(0, _D), pl.ds(0, _BLK)], stags[ob], osems[ob]
        ).wait()

    for n in range(_NG - 1):
        fire(n, n)

    @pl.loop(0, _NU // _NG)
    def _units(i):
        n0 = _NG * i
        for j in range(_NG):
            n = n0 + j

            @pl.when(n + _NG - 1 < _NU)
            def _():
                fire(n + _NG - 1, (j + _NG - 1) % _NG)

            @pl.when(n >= _NO)
            def _():
                wait_out(j % _NO)

            process(n, j, j % _NO)

    for ob in range(_NO):
        wait_out((_NU + ob) % _NO)


def _sc_gather(x_t, t2):
    mesh = plsc.VectorSubcoreMesh(core_axis_name="c", subcore_axis_name="s")
    return pl.kernel(
        _sc_body,
        out_type=jax.ShapeDtypeStruct((_S, _D, _B), jnp.float32),
        mesh=mesh,
        scratch_types=[
            pltpu.VMEM((28, 8, _BLK), jnp.int32),
            [pltpu.VMEM((_BLK, 128), jnp.float32) for _ in range(_NG)],
            [pltpu.VMEM((_D, _BLK), jnp.float32) for _ in range(_NO)],
            [pltpu.SemaphoreType.DMA for _ in range(_NG)],
            [pltpu.SemaphoreType.DMA for _ in range(_NO)],
        ],
        compiler_params=pltpu.CompilerParams(
            use_tc_tiling_on_sc=True, needs_layout_passes=False
        ),
    )(x_t, t2)


@jax.jit
def _emb_lookup(x, table):
    x_t = x.astype(jnp.int32).T
    t2 = _build_t2(table.T)
    out_t = _sc_gather(x_t, t2)
    return jnp.transpose(out_t, (2, 0, 1))


def kernel(x, table):
    return _emb_lookup(x, table)


# speedup vs baseline: 1.1000x; 1.1000x over previous
import functools
import math

import jax
import jax.numpy as jnp
from jax import lax
from jax.experimental import pallas as pl
from jax.experimental.pallas import tpu as pltpu
from jax.experimental.pallas import tpu_sc as plsc

_D = 64
_SCALE = math.sqrt(_D)
_V = 1000000
_B = 16384
_S = 50
_L = 16
_BLK = 128
_NBB = _B // _BLK
_TC_CB = 8192


def _tc_transpose_body(t_ref, out_ref):
    eye = jnp.eye(_D, dtype=jnp.float32) * _SCALE
    out_ref[:, :_D] = lax.dot_general(
        t_ref[...], eye, (((0,), (0,)), ((), ())),
        preferred_element_type=jnp.float32,
    )


def _build_t2(table_t):
    grid = (_V + _TC_CB - 1) // _TC_CB
    return pl.pallas_call(
        _tc_transpose_body,
        grid=(grid,),
        in_specs=[pl.BlockSpec((_D, _TC_CB), lambda j: (0, j))],
        out_specs=pl.BlockSpec((_TC_CB, 128), lambda j: (j, 0)),
        out_shape=jax.ShapeDtypeStruct((_V, 128), jnp.float32),
    )(table_t)


_NG = 4
_NO = 2
_NU = 200


def _sc_body(x_t_hbm, t2_hbm, out_hbm, idx_v, g_bufs, stags, gsems, osems):
    ncores = 2
    wid = lax.axis_index("s") * ncores + lax.axis_index("c")
    bb0 = wid * 4

    for t in range(28):
        bi, stile = t // 7, t % 7
        rows = 8 if stile < 6 else 2
        pltpu.async_copy(
            x_t_hbm.at[pl.ds(stile * 8, rows),
                       pl.ds((bb0 + bi) * _BLK, _BLK)],
            idx_v.at[t, pl.ds(0, rows), :],
            gsems[0],
        )
    for t in range(28):
        rows = 8 if (t % 7) != 6 else 2
        pltpu.make_async_copy(
            x_t_hbm.at[pl.ds(0, rows), pl.ds(0, _BLK)],
            idx_v.at[0, pl.ds(0, rows), :],
            gsems[0],
        ).wait()

    iota = lax.iota(jnp.int32, _L)
    rowvecs = [iota + (g * _L) for g in range(8)]

    def unit_coords(n):
        bi = n // _S
        s = n % _S
        t = bi * 7 + s // 8
        si = s % 8
        return bi, s, t, si

    def fire(n, b):
        _, _, t, si = unit_coords(n)
        pltpu.async_copy(t2_hbm.at[idx_v.at[t, si]], g_bufs[b], gsems[b])

    def process(n, b, ob):
        bi, s, _, _ = unit_coords(n)
        pltpu.make_async_copy(
            t2_hbm.at[pl.ds(0, _BLK)], g_bufs[b], gsems[b]
        ).wait()

        @plsc.parallel_loop(0, _D, unroll=4)
        def _col(k):
            colvec = jnp.full((_L,), 0, jnp.int32) + k
            for g in range(8):
                stags[ob][k, pl.ds(g * _L, _L)] = plsc.load_gather(
                    g_bufs[b], [rowvecs[g], colvec]
                )

        pltpu.async_copy(
            stags[ob], out_hbm.at[s, :, pl.ds((bb0 + bi) * _BLK, _BLK)],
            osems[ob],
        )

    def wait_out(ob):
        pltpu.make_async_copy(
            t2_hbm.at[pl.ds(0, _D), pl.ds(0, _BLK)], stags[ob], osems[ob]
        ).wait()

    for n in range(_NG - 1):
        fire(n, n)

    @pl.loop(0, _NU // _NG)
    def _units(i):
        n0 = _NG * i
        for j in range(_NG):
            n = n0 + j

            @pl.when(n + _NG - 1 < _NU)
            def _():
                fire(n + _NG - 1, (j + _NG - 1) % _NG)

            @pl.when(n >= _NO)
            def _():
                wait_out(j % _NO)

            process(n, j, j % _NO)

    for ob in range(_NO):
        wait_out((_NU + ob) % _NO)


def _sc_gather(x_t, t2):
    mesh = plsc.VectorSubcoreMesh(core_axis_name="c", subcore_axis_name="s")
    return pl.kernel(
        _sc_body,
        out_type=jax.ShapeDtypeStruct((_S, _D, _B), jnp.float32),
        mesh=mesh,
        scratch_types=[
            pltpu.VMEM((28, 8, _BLK), jnp.int32),
            [pltpu.VMEM((_BLK, 128), jnp.float32) for _ in range(_NG)],
            [pltpu.VMEM((_D, _BLK), jnp.float32) for _ in range(_NO)],
            [pltpu.SemaphoreType.DMA for _ in range(_NG)],
            [pltpu.SemaphoreType.DMA for _ in range(_NO)],
        ],
        compiler_params=pltpu.CompilerParams(
            use_tc_tiling_on_sc=True, needs_layout_passes=False
        ),
    )(x_t, t2)


@jax.jit
def _emb_lookup(x, table):
    x_t = x.astype(jnp.int32).T
    t2 = _build_t2(table.T)
    out_t = _sc_gather(x_t, t2)
    return jnp.transpose(out_t, (2, 0, 1))


def kernel(x, table):
    return _emb_lookup(x, table)


# speedup vs baseline: 1.1290x; 1.0264x over previous
import functools
import math

import jax
import jax.numpy as jnp
from jax import lax
from jax.experimental import pallas as pl
from jax.experimental.pallas import tpu as pltpu
from jax.experimental.pallas import tpu_sc as plsc

_D = 64
_SCALE = math.sqrt(_D)
_V = 1000000
_B = 16384
_S = 50
_L = 16
_BLK = 128
_NBB = _B // _BLK
_TC_CB = 16384


def _tc_transpose_body(t_ref, out_ref):
    eye = jnp.eye(_D, dtype=jnp.float32) * _SCALE
    out_ref[:, :_D] = lax.dot_general(
        t_ref[...], eye, (((0,), (0,)), ((), ())),
        preferred_element_type=jnp.float32,
    )


def _build_t2(table_t):
    grid = (_V + _TC_CB - 1) // _TC_CB
    return pl.pallas_call(
        _tc_transpose_body,
        grid=(grid,),
        in_specs=[pl.BlockSpec((_D, _TC_CB), lambda j: (0, j))],
        out_specs=pl.BlockSpec((_TC_CB, 128), lambda j: (j, 0)),
        out_shape=jax.ShapeDtypeStruct((_V, 128), jnp.float32),
    )(table_t)


_NG = 4
_NO = 2
_NU = 200


def _sc_body(x_t_hbm, t2_hbm, out_hbm, idx_v, g_bufs, stags, gsems, osems):
    ncores = 2
    wid = lax.axis_index("s") * ncores + lax.axis_index("c")
    bb0 = wid * 4

    for t in range(28):
        bi, stile = t // 7, t % 7
        rows = 8 if stile < 6 else 2
        pltpu.async_copy(
            x_t_hbm.at[pl.ds(stile * 8, rows),
                       pl.ds((bb0 + bi) * _BLK, _BLK)],
            idx_v.at[t, pl.ds(0, rows), :],
            gsems[0],
        )
    for t in range(28):
        rows = 8 if (t % 7) != 6 else 2
        pltpu.make_async_copy(
            x_t_hbm.at[pl.ds(0, rows), pl.ds(0, _BLK)],
            idx_v.at[0, pl.ds(0, rows), :],
            gsems[0],
        ).wait()

    iota = lax.iota(jnp.int32, _L)
    rowvecs = [iota + (g * _L) for g in range(8)]

    def unit_coords(n):
        bi = n // _S
        s = n % _S
        t = bi * 7 + s // 8
        si = s % 8
        return bi, s, t, si

    def fire(n, b):
        _, _, t, si = unit_coords(n)
        pltpu.async_copy(t2_hbm.at[idx_v.at[t, si]], g_bufs[b], gsems[b])

    def process(n, b, ob):
        bi, s, _, _ = unit_coords(n)
        pltpu.make_async_copy(
            t2_hbm.at[pl.ds(0, _BLK)], g_bufs[b], gsems[b]
        ).wait()

        @plsc.parallel_loop(0, _D, unroll=4)
        def _col(k):
            colvec = jnp.full((_L,), 0, jnp.int32) + k
            for g in range(8):
                stags[ob][k, pl.ds(g * _L, _L)] = plsc.load_gather(
                    g_bufs[b], [rowvecs[g], colvec]
                )

        pltpu.async_copy(
            stags[ob], out_hbm.at[s, :, pl.ds((bb0 + bi) * _BLK, _BLK)],
            osems[ob],
        )

    def wait_out(ob):
        pltpu.make_async_copy(
            t2_hbm.at[pl.ds(0, _D), pl.ds(0, _BLK)], stags[ob], osems[ob]
        ).wait()

    for n in range(_NG - 1):
        fire(n, n)

    @pl.loop(0, _NU // _NG)
    def _units(i):
        n0 = _NG * i
        for j in range(_NG):
            n = n0 + j

            @pl.when(n + _NG - 1 < _NU)
            def _():
                fire(n + _NG - 1, (j + _NG - 1) % _NG)

            @pl.when(n >= _NO)
            def _():
                wait_out(j % _NO)

            process(n, j, j % _NO)

    for ob in range(_NO):
        wait_out((_NU + ob) % _NO)


def _sc_gather(x_t, t2):
    mesh = plsc.VectorSubcoreMesh(core_axis_name="c", subcore_axis_name="s")
    return pl.kernel(
        _sc_body,
        out_type=jax.ShapeDtypeStruct((_S, _D, _B), jnp.float32),
        mesh=mesh,
        scratch_types=[
            pltpu.VMEM((28, 8, _BLK), jnp.int32),
            [pltpu.VMEM((_BLK, 128), jnp.float32) for _ in range(_NG)],
            [pltpu.VMEM((_D, _BLK), jnp.float32) for _ in range(_NO)],
            [pltpu.SemaphoreType.DMA for _ in range(_NG)],
            [pltpu.SemaphoreType.DMA for _ in range(_NO)],
        ],
        compiler_params=pltpu.CompilerParams(
            use_tc_tiling_on_sc=True, needs_layout_passes=False
        ),
    )(x_t, t2)


@jax.jit
def _emb_lookup(x, table):
    x_t = x.astype(jnp.int32).T
    t2 = _build_t2(table.T)
    out_t = _sc_gather(x_t, t2)
    return jnp.transpose(out_t, (2, 0, 1))


def kernel(x, table):
    return _emb_lookup(x, table)


# speedup vs baseline: 1.1377x; 1.0076x over previous
import functools
import math

import jax
import jax.numpy as jnp
from jax import lax
from jax.experimental import pallas as pl
from jax.experimental.pallas import tpu as pltpu
from jax.experimental.pallas import tpu_sc as plsc

_D = 64
_SCALE = math.sqrt(_D)
_V = 1000000
_B = 16384
_S = 50
_L = 16
_BLK = 128
_NBB = _B // _BLK
_TC_CB = 32768


def _tc_transpose_body(t_ref, out_ref):
    eye = jnp.eye(_D, dtype=jnp.float32) * _SCALE
    out_ref[:, :_D] = lax.dot_general(
        t_ref[...], eye, (((0,), (0,)), ((), ())),
        preferred_element_type=jnp.float32,
    )


def _build_t2(table_t):
    grid = (_V + _TC_CB - 1) // _TC_CB
    return pl.pallas_call(
        _tc_transpose_body,
        grid=(grid,),
        in_specs=[pl.BlockSpec((_D, _TC_CB), lambda j: (0, j))],
        out_specs=pl.BlockSpec((_TC_CB, 128), lambda j: (j, 0)),
        out_shape=jax.ShapeDtypeStruct((_V, 128), jnp.float32),
    )(table_t)


_NG = 4
_NO = 2
_NU = 200


def _sc_body(x_t_hbm, t2_hbm, out_hbm, idx_v, g_bufs, stags, gsems, osems):
    ncores = 2
    wid = lax.axis_index("s") * ncores + lax.axis_index("c")
    bb0 = wid * 4

    for t in range(28):
        bi, stile = t // 7, t % 7
        rows = 8 if stile < 6 else 2
        pltpu.async_copy(
            x_t_hbm.at[pl.ds(stile * 8, rows),
                       pl.ds((bb0 + bi) * _BLK, _BLK)],
            idx_v.at[t, pl.ds(0, rows), :],
            gsems[0],
        )
    for t in range(28):
        rows = 8 if (t % 7) != 6 else 2
        pltpu.make_async_copy(
            x_t_hbm.at[pl.ds(0, rows), pl.ds(0, _BLK)],
            idx_v.at[0, pl.ds(0, rows), :],
            gsems[0],
        ).wait()

    iota = lax.iota(jnp.int32, _L)
    rowvecs = [iota + (g * _L) for g in range(8)]

    def unit_coords(n):
        bi = n // _S
        s = n % _S
        t = bi * 7 + s // 8
        si = s % 8
        return bi, s, t, si

    def fire(n, b):
        _, _, t, si = unit_coords(n)
        pltpu.async_copy(t2_hbm.at[idx_v.at[t, si]], g_bufs[b], gsems[b])

    def process(n, b, ob):
        bi, s, _, _ = unit_coords(n)
        pltpu.make_async_copy(
            t2_hbm.at[pl.ds(0, _BLK)], g_bufs[b], gsems[b]
        ).wait()

        @plsc.parallel_loop(0, _D, unroll=4)
        def _col(k):
            colvec = jnp.full((_L,), 0, jnp.int32) + k
            for g in range(8):
                stags[ob][k, pl.ds(g * _L, _L)] = plsc.load_gather(
                    g_bufs[b], [rowvecs[g], colvec]
                )

        pltpu.async_copy(
            stags[ob], out_hbm.at[s, :, pl.ds((bb0 + bi) * _BLK, _BLK)],
            osems[ob],
        )

    def wait_out(ob):
        pltpu.make_async_copy(
            t2_hbm.at[pl.ds(0, _D), pl.ds(0, _BLK)], stags[ob], osems[ob]
        ).wait()

    for n in range(_NG - 1):
        fire(n, n)

    @pl.loop(0, _NU // _NG)
    def _units(i):
        n0 = _NG * i
        for j in range(_NG):
            n = n0 + j

            @pl.when(n + _NG - 1 < _NU)
            def _():
                fire(n + _NG - 1, (j + _NG - 1) % _NG)

            @pl.when(n >= _NO)
            def _():
                wait_out(j % _NO)

            process(n, j, j % _NO)

    for ob in range(_NO):
        wait_out((_NU + ob) % _NO)


def _sc_gather(x_t, t2):
    mesh = plsc.VectorSubcoreMesh(core_axis_name="c", subcore_axis_name="s")
    return pl.kernel(
        _sc_body,
        out_type=jax.ShapeDtypeStruct((_S, _D, _B), jnp.float32),
        mesh=mesh,
        scratch_types=[
            pltpu.VMEM((28, 8, _BLK), jnp.int32),
            [pltpu.VMEM((_BLK, 128), jnp.float32) for _ in range(_NG)],
            [pltpu.VMEM((_D, _BLK), jnp.float32) for _ in range(_NO)],
            [pltpu.SemaphoreType.DMA for _ in range(_NG)],
            [pltpu.SemaphoreType.DMA for _ in range(_NO)],
        ],
        compiler_params=pltpu.CompilerParams(
            use_tc_tiling_on_sc=True, needs_layout_passes=False
        ),
    )(x_t, t2)


@jax.jit
def _emb_lookup(x, table):
    x_t = x.astype(jnp.int32).T
    t2 = _build_t2(table.T)
    out_t = _sc_gather(x_t, t2)
    return jnp.transpose(out_t, (2, 0, 1))


def kernel(x, table):
    return _emb_lookup(x, table)


# speedup vs baseline: 1.1381x; 1.0004x over previous
import math

import jax
import jax.numpy as jnp
from jax import lax
from jax.experimental import pallas as pl
from jax.experimental.pallas import tpu as pltpu
from jax.experimental.pallas import tpu_sc as plsc

_D = 64
_SCALE = math.sqrt(_D)
_V = 1000000
_B = 16384
_S = 50
_L = 16
_BLK = 128
_TC_CB = 32768


def _tc_transpose_body(t_ref, out_ref):
    eye = jnp.eye(_D, dtype=jnp.float32) * _SCALE
    out_ref[:, :_D] = lax.dot_general(
        t_ref[...], eye, (((0,), (0,)), ((), ())),
        preferred_element_type=jnp.float32,
    )


def _build_t2(table_t):
    grid = (_V + _TC_CB - 1) // _TC_CB
    return pl.pallas_call(
        _tc_transpose_body,
        grid=(grid,),
        in_specs=[pl.BlockSpec((_D, _TC_CB), lambda j: (0, j))],
        out_specs=pl.BlockSpec((_TC_CB, 128), lambda j: (j, 0)),
        out_shape=jax.ShapeDtypeStruct((_V, 128), jnp.float32),
    )(table_t)


_NG = 4
_NO = 2
_NU = 200


def _sc_body(x_t_hbm, t2_hbm, out_hbm, idx_v, g_bufs, stags, gsems, osems):
    ncores = 2
    wid = lax.axis_index("s") * ncores + lax.axis_index("c")
    bb0 = wid * 4

    for t in range(28):
        bi, stile = t // 7, t % 7
        rows = 8 if stile < 6 else 2
        pltpu.async_copy(
            x_t_hbm.at[pl.ds(stile * 8, rows),
                       pl.ds((bb0 + bi) * _BLK, _BLK)],
            idx_v.at[t, pl.ds(0, rows), :],
            gsems[0],
        )
    for t in range(28):
        rows = 8 if (t % 7) != 6 else 2
        pltpu.make_async_copy(
            x_t_hbm.at[pl.ds(0, rows), pl.ds(0, _BLK)],
            idx_v.at[0, pl.ds(0, rows), :],
            gsems[0],
        ).wait()

    iota = lax.iota(jnp.int32, _L)
    rowvecs = [iota + (g * _L) for g in range(8)]

    def unit_coords(n):
        bi = n // _S
        s = n % _S
        t = bi * 7 + s // 8
        si = s % 8
        return bi, s, t, si

    def fire(n, b):
        _, _, t, si = unit_coords(n)
        pltpu.async_copy(t2_hbm.at[idx_v.at[t, si]], g_bufs[b], gsems[b])

    def process(n, b, ob):
        bi, s, _, _ = unit_coords(n)
        pltpu.make_async_copy(
            t2_hbm.at[pl.ds(0, _BLK)], g_bufs[b], gsems[b]
        ).wait()

        @plsc.parallel_loop(0, _D, unroll=4)
        def _col(k):
            colvec = jnp.full((_L,), 0, jnp.int32) + k
            for g in range(8):
                stags[ob][k, pl.ds(g * _L, _L)] = plsc.load_gather(
                    g_bufs[b], [rowvecs[g], colvec]
                )

        pltpu.async_copy(
            stags[ob], out_hbm.at[s, :, pl.ds((bb0 + bi) * _BLK, _BLK)],
            osems[ob],
        )

    def wait_out(ob):
        pltpu.make_async_copy(
            t2_hbm.at[pl.ds(0, _D), pl.ds(0, _BLK)], stags[ob], osems[ob]
        ).wait()

    for n in range(_NG - 1):
        fire(n, n)

    @pl.loop(0, _NU // _NG)
    def _units(i):
        n0 = _NG * i
        for j in range(_NG):
            n = n0 + j

            @pl.when(n + _NG - 1 < _NU)
            def _():
                fire(n + _NG - 1, (j + _NG - 1) % _NG)

            @pl.when(n >= _NO)
            def _():
                wait_out(j % _NO)

            process(n, j, j % _NO)

    for ob in range(_NO):
        wait_out((_NU + ob) % _NO)


def _sc_gather(x_t, t2):
    mesh = plsc.VectorSubcoreMesh(core_axis_name="c", subcore_axis_name="s")
    return pl.kernel(
        _sc_body,
        out_type=jax.ShapeDtypeStruct((_S, _D, _B), jnp.float32),
        mesh=mesh,
        scratch_types=[
            pltpu.VMEM((28, 8, _BLK), jnp.int32),
            [pltpu.VMEM((_BLK, 128), jnp.float32) for _ in range(_NG)],
            [pltpu.VMEM((_D, _BLK), jnp.float32) for _ in range(_NO)],
            [pltpu.SemaphoreType.DMA for _ in range(_NG)],
            [pltpu.SemaphoreType.DMA for _ in range(_NO)],
        ],
        compiler_params=pltpu.CompilerParams(
            use_tc_tiling_on_sc=True, needs_layout_passes=False
        ),
    )(x_t, t2)


@jax.jit
def _emb_lookup(x, table):
    x_t = x.astype(jnp.int32).T
    t2 = _build_t2(table.T)
    out_t = _sc_gather(x_t, t2)
    return jnp.transpose(out_t, (2, 0, 1))


def kernel(x, table):
    return _emb_lookup(x, table)


# speedup vs baseline: 1.1405x; 1.0021x over previous
import math

import jax
import jax.numpy as jnp
from jax import lax
from jax.experimental import pallas as pl
from jax.experimental.pallas import tpu as pltpu
from jax.experimental.pallas import tpu_sc as plsc

_D = 64
_SCALE = math.sqrt(_D)
_V = 1000000
_B = 16384
_S = 50
_L = 16
_BLK = 128
_TC_CB = 32768


def _tc_transpose_body(t_ref, out_ref):
    eye = jnp.eye(_D, dtype=jnp.float32) * _SCALE
    out_ref[:, :_D] = lax.dot_general(
        t_ref[...], eye, (((0,), (0,)), ((), ())),
        preferred_element_type=jnp.float32,
    )


def _build_t2(table_t):
    grid = (_V + _TC_CB - 1) // _TC_CB
    return pl.pallas_call(
        _tc_transpose_body,
        grid=(grid,),
        in_specs=[pl.BlockSpec((_D, _TC_CB), lambda j: (0, j))],
        out_specs=pl.BlockSpec((_TC_CB, 128), lambda j: (j, 0)),
        out_shape=jax.ShapeDtypeStruct((_V, 128), jnp.float32),
    )(table_t)


_NG = 4
_NO = 2
_NU = 200


def _sc_body(x_t_hbm, t2_hbm, out_hbm, idx_v, g_bufs, stags, gsems, osems):
    ncores = 2
    wid = lax.axis_index("s") * ncores + lax.axis_index("c")
    bb0 = wid * 4

    for t in range(28):
        bi, stile = t // 7, t % 7
        rows = 8 if stile < 6 else 2
        pltpu.async_copy(
            x_t_hbm.at[pl.ds(stile * 8, rows),
                       pl.ds((bb0 + bi) * _BLK, _BLK)],
            idx_v.at[t, pl.ds(0, rows), :],
            gsems[0],
        )
    for t in range(28):
        rows = 8 if (t % 7) != 6 else 2
        pltpu.make_async_copy(
            x_t_hbm.at[pl.ds(0, rows), pl.ds(0, _BLK)],
            idx_v.at[0, pl.ds(0, rows), :],
            gsems[0],
        ).wait()

    iota = lax.iota(jnp.int32, _L)
    rowvecs = [iota + (g * _L) for g in range(8)]

    def unit_coords(n):
        bi = n % 4
        s = n // 4
        t = bi * 7 + s // 8
        si = s % 8
        return bi, s, t, si

    def fire(n, b):
        _, _, t, si = unit_coords(n)
        pltpu.async_copy(t2_hbm.at[idx_v.at[t, si]], g_bufs[b], gsems[b])

    def process(n, b, ob):
        bi, s, _, _ = unit_coords(n)
        pltpu.make_async_copy(
            t2_hbm.at[pl.ds(0, _BLK)], g_bufs[b], gsems[b]
        ).wait()

        @plsc.parallel_loop(0, _D, unroll=4)
        def _col(k):
            colvec = jnp.full((_L,), 0, jnp.int32) + k
            for g in range(8):
                stags[ob][k, pl.ds(g * _L, _L)] = plsc.load_gather(
                    g_bufs[b], [rowvecs[g], colvec]
                )

        pltpu.async_copy(
            stags[ob], out_hbm.at[s, :, pl.ds((bb0 + bi) * _BLK, _BLK)],
            osems[ob],
        )

    def wait_out(ob):
        pltpu.make_async_copy(
            t2_hbm.at[pl.ds(0, _D), pl.ds(0, _BLK)], stags[ob], osems[ob]
        ).wait()

    for n in range(_NG - 1):
        fire(n, n)

    @pl.loop(0, _NU // _NG)
    def _units(i):
        n0 = _NG * i
        for j in range(_NG):
            n = n0 + j

            @pl.when(n + _NG - 1 < _NU)
            def _():
                fire(n + _NG - 1, (j + _NG - 1) % _NG)

            @pl.when(n >= _NO)
            def _():
                wait_out(j % _NO)

            process(n, j, j % _NO)

    for ob in range(_NO):
        wait_out((_NU + ob) % _NO)


def _sc_gather(x_t, t2):
    mesh = plsc.VectorSubcoreMesh(core_axis_name="c", subcore_axis_name="s")
    return pl.kernel(
        _sc_body,
        out_type=jax.ShapeDtypeStruct((_S, _D, _B), jnp.float32),
        mesh=mesh,
        scratch_types=[
            pltpu.VMEM((28, 8, _BLK), jnp.int32),
            [pltpu.VMEM((_BLK, 128), jnp.float32) for _ in range(_NG)],
            [pltpu.VMEM((_D, _BLK), jnp.float32) for _ in range(_NO)],
            [pltpu.SemaphoreType.DMA for _ in range(_NG)],
            [pltpu.SemaphoreType.DMA for _ in range(_NO)],
        ],
        compiler_params=pltpu.CompilerParams(
            use_tc_tiling_on_sc=True, needs_layout_passes=False
        ),
    )(x_t, t2)


@jax.jit
def _emb_lookup(x, table):
    x_t = x.astype(jnp.int32).T
    t2 = _build_t2(table.T)
    out_t = _sc_gather(x_t, t2)
    return jnp.transpose(out_t, (2, 0, 1))


def kernel(x, table):
    return _emb_lookup(x, table)
